# Initial kernel scaffold; baseline (speedup 1.0000x reference)
import jax
import jax.numpy as jnp
from jax.experimental import pallas as pl


def kernel(x):
    raise NotImplementedError("write your pallas kernel here")


# speedup vs baseline: 2.9561x; 2.9561x over previous
import jax
import jax.numpy as jnp
import numpy as np
from jax.experimental import pallas as pl

B, S, E = 4, 4096, 16

_KS0 = np.uint32(0)
_KS1 = np.uint32(42)
_KS2 = np.uint32(0 ^ 42 ^ 0x1BD11BDA)
_ROT = [[13, 15, 26, 6], [17, 29, 16, 24]]
_KSCHED = [_KS0, _KS1, _KS2]


def _threefry_bits(n):
    x0 = jnp.full(n.shape, _KS0, dtype=jnp.uint32)
    x1 = n + _KS1
    for i in range(5):
        for r in _ROT[i % 2]:
            x0 = x0 + x1
            x1 = (x1 << np.uint32(r)) | (x1 >> np.uint32(32 - r))
            x1 = x0 ^ x1
        x0 = x0 + _KSCHED[(i + 1) % 3]
        x1 = x1 + _KSCHED[(i + 2) % 3] + np.uint32(i + 1)
    return x0 ^ x1


def _router_kernel(oh_ref, ones_ref):
    b_i = jax.lax.broadcasted_iota(jnp.uint32, (B, E, S), 0)
    e_i = jax.lax.broadcasted_iota(jnp.uint32, (B, E, S), 1)
    s_i = jax.lax.broadcasted_iota(jnp.uint32, (B, E, S), 2)
    n = b_i * np.uint32(S * E) + s_i * np.uint32(E) + e_i
    bits = (_threefry_bits(n) >> np.uint32(9)).astype(jnp.int32)
    mx = jnp.max(bits, axis=1, keepdims=True)
    oh = (bits == mx).astype(jnp.float32)
    oh_ref[...] = jnp.swapaxes(oh, 1, 2)
    ones_ref[...] = jnp.ones((B, S, 1), dtype=jnp.float32)


def kernel(x):
    del x
    one_hot, ones = pl.pallas_call(
        _router_kernel,
        out_shape=(
            jax.ShapeDtypeStruct((B, S, E), jnp.float32),
            jax.ShapeDtypeStruct((B, S, 1), jnp.float32),
        ),
    )()
    return (one_hot, ones, one_hot)
